# baseline (speedup 1.0000x reference)
import functools

import jax
import jax.numpy as jnp
import numpy as np
from jax import lax
from jax.experimental import pallas as pl
from jax.experimental.pallas import tpu as pltpu
from jax.experimental.pallas import tpu_sc as plsc

_N_USERS = 20000
_N_ITEMS = 30000
_N = _N_USERS + _N_ITEMS
_D = 64
_HALF = 32
_PAIRS = _HALF // 2
_LAYERS = 3
_E = 800000

_NSUB = 16
_NPAD = 51200
_IDXW = 128
_GRP = 512
_CPG = _GRP // _IDXW
_EPS = 51200
_EPAD = _EPS * _NSUB
_NGRP = _EPS // _GRP
_NCHUNK = _EPS // _IDXW
_ROWS_PER_SUB = _NPAD // _NSUB
_ZCOPIES = _ROWS_PER_SUB // _IDXW

_HI_MASK = np.int32(-65536)
_RN = np.int32(0x8000)

_mesh = plsc.VectorSubcoreMesh(core_axis_name="c", subcore_axis_name="s")


@jax.jit
def _sc_propagate(ego0p, edata):
    @functools.partial(
        pl.kernel,
        out_type=jax.ShapeDtypeStruct((_LAYERS, 2, _NPAD, _PAIRS), jnp.int32),
        mesh=_mesh,
        scratch_types=[
            pltpu.VMEM((2, 3 * _CPG, _IDXW), jnp.int32),
            pltpu.VMEM((4, _IDXW, _HALF), jnp.float32),
            pltpu.VMEM((4, _IDXW, _PAIRS), jnp.int32),
            pltpu.VMEM_SHARED((_NPAD, _HALF), jnp.float32),
            pltpu.SemaphoreType.DMA,
            [pltpu.SemaphoreType.DMA] * 4,
            [pltpu.SemaphoreType.DMA] * 4,
        ],
        compiler_params=pltpu.CompilerParams(use_tc_tiling_on_sc=False,
                                             needs_layout_passes=False),
    )
    def k(ego0_hbm, edata_hbm, out_hbm, ibuf, rows, prow, acc, isem, gsems, ssems):
        c = lax.axis_index("c")
        s = lax.axis_index("s")

        def zero_acc_range():
            @pl.loop(0, _IDXW)
            def _(i):
                rows[0, i, pl.ds(0, 16)] = jnp.zeros((16,), jnp.float32)
                rows[0, i, pl.ds(16, 16)] = jnp.zeros((16,), jnp.float32)

            @pl.loop(0, _ZCOPIES)
            def _(i):
                pltpu.sync_copy(rows.at[0],
                                acc.at[pl.ds(s * _ROWS_PER_SUB + i * _IDXW, _IDXW)])

        def scale_chunk(islot, r):
            @pl.loop(0, _IDXW, step=16)
            def _(i0):
                w16 = plsc.bitcast(ibuf[islot, 2 * _CPG + r, pl.ds(i0, 16)],
                                   jnp.float32)
                for e in range(16):
                    wv = w16[e]
                    v = prow[r, i0 + e, pl.ds(0, 16)]
                    lo = plsc.bitcast(lax.shift_left(v, 16), jnp.float32)
                    hi = plsc.bitcast(lax.bitwise_and(v, _HI_MASK), jnp.float32)
                    rows[r, i0 + e, pl.ds(0, 16)] = lo * wv
                    rows[r, i0 + e, pl.ds(16, 16)] = hi * wv

        zero_acc_range()
        plsc.subcore_barrier()

        for layer in range(_LAYERS):
            gsrc = ego0_hbm.at[c] if layer == 0 else out_hbm.at[layer - 1, c]

            def fire_gather(islot, j, slot):
                return pltpu.async_copy(gsrc.at[ibuf.at[islot, j]],
                                        prow.at[slot], gsems[slot])

            def fire_scatter(islot, r):
                return pltpu.async_copy(rows.at[r],
                                        acc.at[ibuf.at[islot, _CPG + r]],
                                        ssems[r], add=True)

            def drain_scatter(slot):
                pltpu.make_async_copy(rows.at[slot], acc.at[pl.ds(0, _IDXW)],
                                      ssems[slot]).wait()

            def wait_gather(slot):
                pltpu.make_async_copy(gsrc.at[pl.ds(0, _IDXW)], prow.at[slot],
                                      gsems[slot]).wait()

            pltpu.sync_copy(edata_hbm.at[s * _NGRP], ibuf.at[0])
            fire_gather(0, 0, 0)
            fire_gather(0, 1, 1)

            @pl.loop(0, _NGRP // 2)
            def _(t2):
                for half in range(2):
                    g = 2 * t2 + half
                    for r in range(_CPG):
                        t = 4 * g + r
                        wait_gather(r)
                        scale_chunk(half, r)
                        fire_scatter(half, r)

                        @pl.when(t >= 2)
                        def _():
                            drain_scatter((r + 2) % 4)

                        if r == 1:
                            @pl.when(g < _NGRP - 1)
                            def _():
                                pltpu.async_copy(edata_hbm.at[s * _NGRP + g + 1],
                                                 ibuf.at[1 - half], isem)

                        if r == 2:
                            @pl.when(g < _NGRP - 1)
                            def _():
                                pltpu.make_async_copy(
                                    edata_hbm.at[s * _NGRP],
                                    ibuf.at[1 - half], isem).wait()

                        @pl.when(t + 2 < _NCHUNK)
                        def _():
                            if r < 2:
                                fire_gather(half, r + 2, (r + 2) % 4)
                            else:
                                fire_gather(1 - half, r - 2, (r + 2) % 4)

            drain_scatter(2)
            drain_scatter(3)

            plsc.subcore_barrier()

            @pl.loop(0, _ZCOPIES)
            def _(i):
                r0 = s * _ROWS_PER_SUB + i * _IDXW
                pltpu.sync_copy(acc.at[pl.ds(r0, _IDXW)], rows.at[1])

                @pl.loop(0, _IDXW)
                def _(q):
                    lo = plsc.bitcast(rows[1, q, pl.ds(0, 16)], jnp.int32)
                    hi = plsc.bitcast(rows[1, q, pl.ds(16, 16)], jnp.int32)
                    lo_r = lax.shift_right_logical(lo + _RN, 16)
                    hi_r = lax.bitwise_and(hi + _RN, _HI_MASK)
                    prow[0, q, pl.ds(0, 16)] = lax.bitwise_or(lo_r, hi_r)

                pltpu.sync_copy(prow.at[0], out_hbm.at[layer, c, pl.ds(r0, _IDXW)])
            if layer < _LAYERS - 1:
                zero_acc_range()

            plsc.subcore_barrier()

    return k(ego0p, edata)


_BN = 2000


def _mean_body(ego0_ref, layers_ref, o_ref):
    l0 = layers_ref[0].astype(jnp.float32)
    l1 = layers_ref[1].astype(jnp.float32)
    l2 = layers_ref[2].astype(jnp.float32)
    s0 = ego0_ref[0] + l0[0] + l1[0] + l2[0]
    s1 = ego0_ref[1] + l0[1] + l1[1] + l2[1]
    o_ref[:, 0:_HALF] = s0 * 0.25
    o_ref[:, _HALF:_D] = s1 * 0.25


@jax.jit
def _mean(ego0, layers):
    return pl.pallas_call(
        _mean_body,
        out_shape=jax.ShapeDtypeStruct((_N, _D), jnp.float32),
        grid=(_N // _BN,),
        in_specs=[
            pl.BlockSpec((2, _BN, _HALF), lambda i: (0, i, 0)),
            pl.BlockSpec((_LAYERS, 2, _BN, _HALF), lambda i: (0, 0, i, 0)),
        ],
        out_specs=pl.BlockSpec((_BN, _D), lambda i: (i, 0)),
    )(ego0, layers)


def kernel(edge_index, edge_weight, user_table, item_table):
    ego0 = jnp.concatenate([user_table, item_table], axis=0)
    ego0 = jnp.pad(ego0, ((0, _NPAD - _N), (0, 0)))
    ego0_split = ego0.reshape(_NPAD, 2, _HALF).transpose(1, 0, 2)
    ego0_packed = lax.bitcast_convert_type(
        ego0_split.astype(jnp.bfloat16).reshape(2, _NPAD, _PAIRS, 2), jnp.int32)
    pad = _EPAD - _E
    src = jnp.pad(edge_index[0], (0, pad)).reshape(-1, _CPG, _IDXW)
    dst = jnp.pad(edge_index[1], (0, pad)).reshape(-1, _CPG, _IDXW)
    wbits = lax.bitcast_convert_type(
        jnp.pad(edge_weight, (0, pad)), jnp.int32).reshape(-1, _CPG, _IDXW)
    edata = jnp.concatenate([src, dst, wbits], axis=1)
    layers_packed = _sc_propagate(ego0_packed, edata)
    layers = lax.bitcast_convert_type(
        layers_packed, jnp.bfloat16).reshape(_LAYERS, 2, _NPAD, _HALF)
    final = _mean(ego0_split, layers)
    return final[:_N_USERS], final[_N_USERS:]


# speedup vs baseline: 1.1432x; 1.1432x over previous
import functools

import jax
import jax.numpy as jnp
import numpy as np
from jax import lax
from jax.experimental import pallas as pl
from jax.experimental.pallas import tpu as pltpu
from jax.experimental.pallas import tpu_sc as plsc

_N_USERS = 20000
_N_ITEMS = 30000
_N = _N_USERS + _N_ITEMS
_D = 64
_HALF = 32
_PAIRS = _HALF // 2
_LAYERS = 3
_E = 800000

_NSUB = 16
_NPAD = 51200
_IDXW = 128
_GRP = 512
_CPG = _GRP // _IDXW
_EPS = 51200
_EPAD = _EPS * _NSUB
_NGRP = _EPS // _GRP
_NCHUNK = _EPS // _IDXW
_ROWS_PER_SUB = _NPAD // _NSUB
_ZCOPIES = _ROWS_PER_SUB // _IDXW

_HI_MASK = np.int32(-65536)
_RN = np.int32(0x8000)

_mesh = plsc.VectorSubcoreMesh(core_axis_name="c", subcore_axis_name="s")


@jax.jit
def _sc_propagate(ego0p, edata):
    @functools.partial(
        pl.kernel,
        out_type=jax.ShapeDtypeStruct((_LAYERS, 2, _NPAD, _PAIRS), jnp.int32),
        mesh=_mesh,
        scratch_types=[
            pltpu.VMEM((2, 3 * _CPG, _IDXW), jnp.int32),
            pltpu.VMEM((4, _IDXW, _HALF), jnp.float32),
            pltpu.VMEM((4, _IDXW, _PAIRS), jnp.int32),
            pltpu.VMEM_SHARED((_NPAD, _HALF), jnp.float32),
            pltpu.SemaphoreType.DMA,
            [pltpu.SemaphoreType.DMA] * 4,
            [pltpu.SemaphoreType.DMA] * 4,
        ],
        compiler_params=pltpu.CompilerParams(use_tc_tiling_on_sc=False,
                                             needs_layout_passes=False),
    )
    def k(ego0_hbm, edata_hbm, out_hbm, ibuf, rows, prow, acc, isem, gsems, ssems):
        c = lax.axis_index("c")
        s = lax.axis_index("s")

        def zero_acc_range():
            @pl.loop(0, _IDXW)
            def _(i):
                rows[0, i, pl.ds(0, 16)] = jnp.zeros((16,), jnp.float32)
                rows[0, i, pl.ds(16, 16)] = jnp.zeros((16,), jnp.float32)

            @pl.loop(0, _ZCOPIES)
            def _(i):
                pltpu.sync_copy(rows.at[0],
                                acc.at[pl.ds(s * _ROWS_PER_SUB + i * _IDXW, _IDXW)])

        def scale_chunk(islot, r):
            @plsc.parallel_loop(0, _IDXW, step=16)
            def _(i0):
                w16 = plsc.bitcast(ibuf[islot, 2 * _CPG + r, pl.ds(i0, 16)],
                                   jnp.float32)
                for e in range(16):
                    wv = w16[e]
                    v = prow[r, i0 + e, pl.ds(0, 16)]
                    lo = plsc.bitcast(lax.shift_left(v, 16), jnp.float32)
                    hi = plsc.bitcast(lax.bitwise_and(v, _HI_MASK), jnp.float32)
                    rows[r, i0 + e, pl.ds(0, 16)] = lo * wv
                    rows[r, i0 + e, pl.ds(16, 16)] = hi * wv

        zero_acc_range()
        plsc.subcore_barrier()

        for layer in range(_LAYERS):
            gsrc = ego0_hbm.at[c] if layer == 0 else out_hbm.at[layer - 1, c]

            def fire_gather(islot, j, slot):
                return pltpu.async_copy(gsrc.at[ibuf.at[islot, j]],
                                        prow.at[slot], gsems[slot])

            def fire_scatter(islot, r):
                return pltpu.async_copy(rows.at[r],
                                        acc.at[ibuf.at[islot, _CPG + r]],
                                        ssems[r], add=True)

            def drain_scatter(slot):
                pltpu.make_async_copy(rows.at[slot], acc.at[pl.ds(0, _IDXW)],
                                      ssems[slot]).wait()

            def wait_gather(slot):
                pltpu.make_async_copy(gsrc.at[pl.ds(0, _IDXW)], prow.at[slot],
                                      gsems[slot]).wait()

            pltpu.sync_copy(edata_hbm.at[s * _NGRP], ibuf.at[0])
            fire_gather(0, 0, 0)
            fire_gather(0, 1, 1)

            @pl.loop(0, _NGRP // 2)
            def _(t2):
                for half in range(2):
                    g = 2 * t2 + half
                    for r in range(_CPG):
                        t = 4 * g + r
                        wait_gather(r)
                        scale_chunk(half, r)
                        fire_scatter(half, r)

                        @pl.when(t >= 2)
                        def _():
                            drain_scatter((r + 2) % 4)

                        if r == 1:
                            @pl.when(g < _NGRP - 1)
                            def _():
                                pltpu.async_copy(edata_hbm.at[s * _NGRP + g + 1],
                                                 ibuf.at[1 - half], isem)

                        if r == 2:
                            @pl.when(g < _NGRP - 1)
                            def _():
                                pltpu.make_async_copy(
                                    edata_hbm.at[s * _NGRP],
                                    ibuf.at[1 - half], isem).wait()

                        @pl.when(t + 2 < _NCHUNK)
                        def _():
                            if r < 2:
                                fire_gather(half, r + 2, (r + 2) % 4)
                            else:
                                fire_gather(1 - half, r - 2, (r + 2) % 4)

            drain_scatter(2)
            drain_scatter(3)

            plsc.subcore_barrier()

            @pl.loop(0, _ZCOPIES)
            def _(i):
                r0 = s * _ROWS_PER_SUB + i * _IDXW
                pltpu.sync_copy(acc.at[pl.ds(r0, _IDXW)], rows.at[1])

                @plsc.parallel_loop(0, _IDXW, step=1)
                def _(q):
                    lo = plsc.bitcast(rows[1, q, pl.ds(0, 16)], jnp.int32)
                    hi = plsc.bitcast(rows[1, q, pl.ds(16, 16)], jnp.int32)
                    lo_r = lax.shift_right_logical(lo + _RN, 16)
                    hi_r = lax.bitwise_and(hi + _RN, _HI_MASK)
                    prow[0, q, pl.ds(0, 16)] = lax.bitwise_or(lo_r, hi_r)

                pltpu.sync_copy(prow.at[0], out_hbm.at[layer, c, pl.ds(r0, _IDXW)])
            if layer < _LAYERS - 1:
                zero_acc_range()

            plsc.subcore_barrier()

    return k(ego0p, edata)


_BN = 2000


def _mean_body(ego0_ref, layers_ref, o_ref):
    l0 = layers_ref[0].astype(jnp.float32)
    l1 = layers_ref[1].astype(jnp.float32)
    l2 = layers_ref[2].astype(jnp.float32)
    s0 = ego0_ref[0] + l0[0] + l1[0] + l2[0]
    s1 = ego0_ref[1] + l0[1] + l1[1] + l2[1]
    o_ref[:, 0:_HALF] = s0 * 0.25
    o_ref[:, _HALF:_D] = s1 * 0.25


@jax.jit
def _mean(ego0, layers):
    return pl.pallas_call(
        _mean_body,
        out_shape=jax.ShapeDtypeStruct((_N, _D), jnp.float32),
        grid=(_N // _BN,),
        in_specs=[
            pl.BlockSpec((2, _BN, _HALF), lambda i: (0, i, 0)),
            pl.BlockSpec((_LAYERS, 2, _BN, _HALF), lambda i: (0, 0, i, 0)),
        ],
        out_specs=pl.BlockSpec((_BN, _D), lambda i: (i, 0)),
    )(ego0, layers)


def kernel(edge_index, edge_weight, user_table, item_table):
    ego0 = jnp.concatenate([user_table, item_table], axis=0)
    ego0 = jnp.pad(ego0, ((0, _NPAD - _N), (0, 0)))
    ego0_split = ego0.reshape(_NPAD, 2, _HALF).transpose(1, 0, 2)
    ego0_packed = lax.bitcast_convert_type(
        ego0_split.astype(jnp.bfloat16).reshape(2, _NPAD, _PAIRS, 2), jnp.int32)
    pad = _EPAD - _E
    src = jnp.pad(edge_index[0], (0, pad)).reshape(-1, _CPG, _IDXW)
    dst = jnp.pad(edge_index[1], (0, pad)).reshape(-1, _CPG, _IDXW)
    wbits = lax.bitcast_convert_type(
        jnp.pad(edge_weight, (0, pad)), jnp.int32).reshape(-1, _CPG, _IDXW)
    edata = jnp.concatenate([src, dst, wbits], axis=1)
    layers_packed = _sc_propagate(ego0_packed, edata)
    layers = lax.bitcast_convert_type(
        layers_packed, jnp.bfloat16).reshape(_LAYERS, 2, _NPAD, _HALF)
    final = _mean(ego0_split, layers)
    return final[:_N_USERS], final[_N_USERS:]


# speedup vs baseline: 1.2043x; 1.0534x over previous
import functools

import jax
import jax.numpy as jnp
import numpy as np
from jax import lax
from jax.experimental import pallas as pl
from jax.experimental.pallas import tpu as pltpu
from jax.experimental.pallas import tpu_sc as plsc

_N_USERS = 20000
_N_ITEMS = 30000
_N = _N_USERS + _N_ITEMS
_D = 64
_HALF = 32
_PAIRS = _HALF // 2
_LAYERS = 3
_E = 800000

_NSUB = 16
_NPAD = 51200
_IDXW = 128
_GRP = 512
_CPG = _GRP // _IDXW
_EPS = 51200
_EPAD = _EPS * _NSUB
_NGRP = _EPS // _GRP
_NCHUNK = _EPS // _IDXW
_ROWS_PER_SUB = _NPAD // _NSUB
_ZCOPIES = _ROWS_PER_SUB // _IDXW

_HI_MASK = np.int32(-65536)
_RN = np.int32(0x8000)

_mesh = plsc.VectorSubcoreMesh(core_axis_name="c", subcore_axis_name="s")


@jax.jit
def _sc_propagate(ego0p, edata):
    @functools.partial(
        pl.kernel,
        out_type=jax.ShapeDtypeStruct((_LAYERS, 2, _NPAD, _PAIRS), jnp.int32),
        mesh=_mesh,
        scratch_types=[
            pltpu.VMEM((2, 3 * _CPG, _IDXW), jnp.int32),
            pltpu.VMEM((4, _IDXW, _HALF), jnp.float32),
            pltpu.VMEM((4, _IDXW, _PAIRS), jnp.int32),
            pltpu.VMEM_SHARED((_NPAD, _HALF), jnp.float32),
            pltpu.SemaphoreType.DMA,
            [pltpu.SemaphoreType.DMA] * 4,
            [pltpu.SemaphoreType.DMA] * 4,
        ],
        compiler_params=pltpu.CompilerParams(use_tc_tiling_on_sc=False,
                                             needs_layout_passes=False),
    )
    def k(ego0_hbm, edata_hbm, out_hbm, ibuf, rows, prow, acc, isem, gsems, ssems):
        c = lax.axis_index("c")
        s = lax.axis_index("s")

        def zero_acc_range():
            @pl.loop(0, _IDXW)
            def _(i):
                rows[0, i, pl.ds(0, 16)] = jnp.zeros((16,), jnp.float32)
                rows[0, i, pl.ds(16, 16)] = jnp.zeros((16,), jnp.float32)

            @pl.loop(0, _ZCOPIES)
            def _(i):
                pltpu.sync_copy(rows.at[0],
                                acc.at[pl.ds(s * _ROWS_PER_SUB + i * _IDXW, _IDXW)])

        def scale_chunk(islot, r):
            @plsc.parallel_loop(0, _IDXW, step=16)
            def _(i0):
                w16 = plsc.bitcast(ibuf[islot, 2 * _CPG + r, pl.ds(i0, 16)],
                                   jnp.float32)
                for e in range(16):
                    wv = w16[e]
                    v = prow[r, i0 + e, pl.ds(0, 16)]
                    lo = plsc.bitcast(lax.shift_left(v, 16), jnp.float32)
                    hi = plsc.bitcast(lax.bitwise_and(v, _HI_MASK), jnp.float32)
                    rows[r, i0 + e, pl.ds(0, 16)] = lo * wv
                    rows[r, i0 + e, pl.ds(16, 16)] = hi * wv

        zero_acc_range()
        plsc.subcore_barrier()

        for layer in range(_LAYERS):
            gsrc = ego0_hbm.at[c] if layer == 0 else out_hbm.at[layer - 1, c]

            def fire_gather(islot, j, slot):
                return pltpu.async_copy(gsrc.at[ibuf.at[islot, j]],
                                        prow.at[slot], gsems[slot])

            def fire_scatter(islot, r):
                return pltpu.async_copy(rows.at[r],
                                        acc.at[ibuf.at[islot, _CPG + r]],
                                        ssems[r], add=True)

            def drain_scatter(slot):
                pltpu.make_async_copy(rows.at[slot], acc.at[pl.ds(0, _IDXW)],
                                      ssems[slot]).wait()

            def wait_gather(slot):
                pltpu.make_async_copy(gsrc.at[pl.ds(0, _IDXW)], prow.at[slot],
                                      gsems[slot]).wait()

            pltpu.sync_copy(edata_hbm.at[s * _NGRP], ibuf.at[0])
            fire_gather(0, 0, 0)
            fire_gather(0, 1, 1)

            @pl.loop(0, _NGRP // 2)
            def _(t2):
                for half in range(2):
                    g = 2 * t2 + half
                    for r in range(_CPG):
                        t = 4 * g + r
                        wait_gather(r)

                        if r == 2:
                            @pl.when(g < _NGRP - 1)
                            def _():
                                pltpu.make_async_copy(
                                    edata_hbm.at[s * _NGRP],
                                    ibuf.at[1 - half], isem).wait()

                        @pl.when(t + 2 < _NCHUNK)
                        def _():
                            if r < 2:
                                fire_gather(half, r + 2, (r + 2) % 4)
                            else:
                                fire_gather(1 - half, r - 2, (r + 2) % 4)

                        @pl.when(t >= 2)
                        def _():
                            drain_scatter((r + 2) % 4)

                        if r == 1:
                            @pl.when(g < _NGRP - 1)
                            def _():
                                pltpu.async_copy(edata_hbm.at[s * _NGRP + g + 1],
                                                 ibuf.at[1 - half], isem)

                        scale_chunk(half, r)
                        fire_scatter(half, r)

            drain_scatter(2)
            drain_scatter(3)

            plsc.subcore_barrier()

            @pl.loop(0, _ZCOPIES)
            def _(i):
                r0 = s * _ROWS_PER_SUB + i * _IDXW
                pltpu.sync_copy(acc.at[pl.ds(r0, _IDXW)], rows.at[1])

                @plsc.parallel_loop(0, _IDXW, step=1)
                def _(q):
                    lo = plsc.bitcast(rows[1, q, pl.ds(0, 16)], jnp.int32)
                    hi = plsc.bitcast(rows[1, q, pl.ds(16, 16)], jnp.int32)
                    lo_r = lax.shift_right_logical(lo + _RN, 16)
                    hi_r = lax.bitwise_and(hi + _RN, _HI_MASK)
                    prow[0, q, pl.ds(0, 16)] = lax.bitwise_or(lo_r, hi_r)

                pltpu.sync_copy(prow.at[0], out_hbm.at[layer, c, pl.ds(r0, _IDXW)])
            if layer < _LAYERS - 1:
                zero_acc_range()

            plsc.subcore_barrier()

    return k(ego0p, edata)


_BN = 2000


def _mean_body(ego0_ref, layers_ref, o_ref):
    l0 = layers_ref[0].astype(jnp.float32)
    l1 = layers_ref[1].astype(jnp.float32)
    l2 = layers_ref[2].astype(jnp.float32)
    s0 = ego0_ref[0] + l0[0] + l1[0] + l2[0]
    s1 = ego0_ref[1] + l0[1] + l1[1] + l2[1]
    o_ref[:, 0:_HALF] = s0 * 0.25
    o_ref[:, _HALF:_D] = s1 * 0.25


@jax.jit
def _mean(ego0, layers):
    return pl.pallas_call(
        _mean_body,
        out_shape=jax.ShapeDtypeStruct((_N, _D), jnp.float32),
        grid=(_N // _BN,),
        in_specs=[
            pl.BlockSpec((2, _BN, _HALF), lambda i: (0, i, 0)),
            pl.BlockSpec((_LAYERS, 2, _BN, _HALF), lambda i: (0, 0, i, 0)),
        ],
        out_specs=pl.BlockSpec((_BN, _D), lambda i: (i, 0)),
    )(ego0, layers)


def kernel(edge_index, edge_weight, user_table, item_table):
    ego0 = jnp.concatenate([user_table, item_table], axis=0)
    ego0 = jnp.pad(ego0, ((0, _NPAD - _N), (0, 0)))
    ego0_split = ego0.reshape(_NPAD, 2, _HALF).transpose(1, 0, 2)
    ego0_packed = lax.bitcast_convert_type(
        ego0_split.astype(jnp.bfloat16).reshape(2, _NPAD, _PAIRS, 2), jnp.int32)
    pad = _EPAD - _E
    src = jnp.pad(edge_index[0], (0, pad)).reshape(-1, _CPG, _IDXW)
    dst = jnp.pad(edge_index[1], (0, pad)).reshape(-1, _CPG, _IDXW)
    wbits = lax.bitcast_convert_type(
        jnp.pad(edge_weight, (0, pad)), jnp.int32).reshape(-1, _CPG, _IDXW)
    edata = jnp.concatenate([src, dst, wbits], axis=1)
    layers_packed = _sc_propagate(ego0_packed, edata)
    layers = lax.bitcast_convert_type(
        layers_packed, jnp.bfloat16).reshape(_LAYERS, 2, _NPAD, _HALF)
    final = _mean(ego0_split, layers)
    return final[:_N_USERS], final[_N_USERS:]
